# Initial kernel scaffold; baseline (speedup 1.0000x reference)
import functools

import jax
import jax.numpy as jnp
from jax import lax
from jax.experimental import pallas as pl
from jax.experimental.pallas import tpu as pltpu
from jax.experimental.pallas import tpu_sc as plsc

N = 10000
D = 128
E = 320000

NC = 2
NS = 16
NW = NC * NS

NP = 10240
ROWS_PER_TILE = NP // NS

K = 128
EPW = 10240
ITERS = EPW // K
EP = NW * EPW

_Z16F = jnp.zeros((16,), jnp.float32)
_O16F = jnp.ones((16,), jnp.float32)


def _zero_vmem_2d(ref, rows, cols):
    def body(i, c):
        for k in range(cols // 16):
            ref[i, pl.ds(k * 16, 16)] = _Z16F
        return c
    lax.fori_loop(0, rows, body, 0)


def _mesh():
    return plsc.VectorSubcoreMesh(
        core_axis_name="c", subcore_axis_name="s", num_cores=NC, num_subcores=NS
    )


def _deg_body(dst_hbm, deg_out, dst_v, ones_v, stage_v, deg_sh):
    c = lax.axis_index("c")
    s = lax.axis_index("s")
    w = c * NS + s

    pltpu.sync_copy(dst_hbm.at[pl.ds(w * ITERS, ITERS)], dst_v)

    def ones_body(i, carry):
        ones_v[pl.ds(i * 16, 16)] = _O16F
        return carry
    lax.fori_loop(0, K // 16, ones_body, 0)

    def zero_body(i, carry):
        stage_v[pl.ds(i * 16, 16)] = _Z16F
        return carry
    lax.fori_loop(0, ROWS_PER_TILE // 16, zero_body, 0)

    pltpu.sync_copy(stage_v, deg_sh.at[pl.ds(s * ROWS_PER_TILE, ROWS_PER_TILE)])
    plsc.subcore_barrier()

    def body(j, carry):
        pltpu.sync_copy(ones_v, deg_sh.at[dst_v.at[j]], add=True)
        return carry
    lax.fori_loop(0, ITERS, body, 0)

    plsc.subcore_barrier()
    pltpu.sync_copy(deg_sh.at[pl.ds(s * ROWS_PER_TILE, ROWS_PER_TILE)], stage_v)
    pltpu.sync_copy(stage_v, deg_out.at[c, pl.ds(s * ROWS_PER_TILE, ROWS_PER_TILE)])


_deg_call = functools.partial(
    pl.kernel,
    out_type=jax.ShapeDtypeStruct((NC, NP), jnp.float32),
    mesh=_mesh(),
    scratch_types=[
        pltpu.VMEM((ITERS, K), jnp.int32),
        pltpu.VMEM((K,), jnp.float32),
        pltpu.VMEM((ROWS_PER_TILE,), jnp.float32),
        pltpu.VMEM_SHARED((NP,), jnp.float32),
    ],
)


def _prop_body(g_hbm, src_hbm, dst_hbm, s_out, src_v, dst_v, buf0, buf1, s_sh,
               sem0, sem1):
    c = lax.axis_index("c")
    s = lax.axis_index("s")
    w = c * NS + s

    pltpu.sync_copy(src_hbm.at[pl.ds(w * ITERS, ITERS)], src_v)
    pltpu.sync_copy(dst_hbm.at[pl.ds(w * ITERS, ITERS)], dst_v)

    _zero_vmem_2d(buf0, K, D)
    for r in range(ROWS_PER_TILE // K):
        pltpu.sync_copy(buf0, s_sh.at[pl.ds(s * ROWS_PER_TILE + r * K, K)])
    plsc.subcore_barrier()

    pltpu.async_copy(g_hbm.at[src_v.at[0]], buf0, sem0)

    def body(i, carry):
        j = i * 2
        pltpu.async_copy(g_hbm.at[src_v.at[j + 1]], buf1, sem1)
        pltpu.make_async_copy(g_hbm.at[src_v.at[j]], buf0, sem0).wait()
        pltpu.sync_copy(buf0, s_sh.at[dst_v.at[j]], add=True)

        @pl.when(j + 2 < ITERS)
        def _():
            pltpu.async_copy(g_hbm.at[src_v.at[j + 2]], buf0, sem0)

        pltpu.make_async_copy(g_hbm.at[src_v.at[j + 1]], buf1, sem1).wait()
        pltpu.sync_copy(buf1, s_sh.at[dst_v.at[j + 1]], add=True)
        return carry

    lax.fori_loop(0, ITERS // 2, body, 0)

    plsc.subcore_barrier()
    for r in range(ROWS_PER_TILE // K):
        off = s * ROWS_PER_TILE + r * K
        pltpu.sync_copy(s_sh.at[pl.ds(off, K)], buf0)
        pltpu.sync_copy(buf0, s_out.at[c, pl.ds(off, K)])


_prop_call = functools.partial(
    pl.kernel,
    out_type=jax.ShapeDtypeStruct((NC, NP, D), jnp.float32),
    mesh=_mesh(),
    scratch_types=[
        pltpu.VMEM((ITERS, K), jnp.int32),
        pltpu.VMEM((ITERS, K), jnp.int32),
        pltpu.VMEM((K, D), jnp.float32),
        pltpu.VMEM((K, D), jnp.float32),
        pltpu.VMEM_SHARED((NP, D), jnp.float32),
        pltpu.SemaphoreType.DMA,
        pltpu.SemaphoreType.DMA,
    ],
)


R = 1024


def _dinv(dA, dB):
    return lax.rsqrt(dA[...] + dB[...] + 1.0)


def _l1_body(x, W, dA, dB, o):
    o[...] = _dinv(dA, dB) * jnp.dot(
        x[...], W[...], preferred_element_type=jnp.float32
    )


def _mid_body(sA, sB, g, dA, dB, b, W, o):
    dinv = _dinv(dA, dB)
    h = jnp.maximum(dinv * (sA[...] + sB[...] + g[...]) + b[...], 0.0)
    o[...] = dinv * jnp.dot(h, W[...], preferred_element_type=jnp.float32)


def _fin_body(sA, sB, g, dA, dB, b, W, bl, o):
    dinv = _dinv(dA, dB)
    h = jnp.maximum(dinv * (sA[...] + sB[...] + g[...]) + b[...], 0.0)
    o[...] = jnp.dot(h, W[...], preferred_element_type=jnp.float32) + bl[...]


_rows = pl.BlockSpec((R, D), lambda i: (i, 0))
_full = pl.BlockSpec((D, D), lambda i: (0, 0))
_col = pl.BlockSpec((R, 1), lambda i: (i, 0))
_row1 = pl.BlockSpec((1, D), lambda i: (0, 0))
_ospec = pl.BlockSpec((R, D), lambda i: (i, 0))
_oshape = jax.ShapeDtypeStruct((NP, D), jnp.float32)
_grid = (NP // R,)

_l1_call = pl.pallas_call(
    _l1_body, grid=_grid, out_shape=_oshape,
    in_specs=[_rows, _full, _col, _col], out_specs=_ospec,
)
_mid_call = pl.pallas_call(
    _mid_body, grid=_grid, out_shape=_oshape,
    in_specs=[_rows, _rows, _rows, _col, _col, _row1, _full], out_specs=_ospec,
)
_fin_call = pl.pallas_call(
    _fin_body, grid=_grid, out_shape=_oshape,
    in_specs=[_rows, _rows, _rows, _col, _col, _row1, _full, _row1],
    out_specs=_ospec,
)


def kernel(x, edge_index, W1, b1, W2, b2, W3, b3, Wl, bl):
    ei = edge_index.astype(jnp.int32)
    pad = EP - E
    srcp = jnp.concatenate([ei[0], jnp.zeros((pad,), jnp.int32)])
    dstp = jnp.concatenate([ei[1], jnp.full((pad,), N, jnp.int32)])
    src2 = srcp.reshape(NW * ITERS, K)
    dst2 = dstp.reshape(NW * ITERS, K)

    xp = jnp.pad(x, ((0, NP - N), (0, 0)))
    b1r = b1.reshape(1, D)
    b2r = b2.reshape(1, D)
    b3r = b3.reshape(1, D)
    blr = bl.reshape(1, D)

    deg = _deg_call(_deg_body)(dst2)
    dA = deg[0].reshape(NP, 1)
    dB = deg[1].reshape(NP, 1)

    g1 = _l1_call(xp, W1, dA, dB)
    s1 = _prop_call(_prop_body)(g1, src2, dst2)
    g2 = _mid_call(s1[0], s1[1], g1, dA, dB, b1r, W2)
    s2 = _prop_call(_prop_body)(g2, src2, dst2)
    g3 = _mid_call(s2[0], s2[1], g2, dA, dB, b2r, W3)
    s3 = _prop_call(_prop_body)(g3, src2, dst2)
    out = _fin_call(s3[0], s3[1], g3, dA, dB, b3r, Wl, blr)
    return out[:N]


# speedup vs baseline: 6.6692x; 6.6692x over previous
import functools

import jax
import jax.numpy as jnp
from jax import lax
from jax.experimental import pallas as pl
from jax.experimental.pallas import tpu as pltpu
from jax.experimental.pallas import tpu_sc as plsc

N = 10000
D = 128
E = 320000

NC = 2
NS = 16
NW = NC * NS

NP = 10240
ROWS_PER_TILE = NP // NS

K = 128
EPW = 10240
ITERS = EPW // K
EP = NW * EPW

def _zero_vmem_2d(ref, rows, cols):
    z = jnp.zeros((16,), jnp.float32)
    def body(i, c):
        for k in range(cols // 16):
            ref[i, pl.ds(k * 16, 16)] = z
        return c
    lax.fori_loop(0, rows, body, 0)


def _mesh():
    return plsc.VectorSubcoreMesh(
        core_axis_name="c", subcore_axis_name="s", num_cores=NC, num_subcores=NS
    )


def _deg_body(dst_hbm, deg_out, dst_v, ones_v, stage_v, deg_sh):
    c = lax.axis_index("c")
    s = lax.axis_index("s")
    w = c * NS + s

    pltpu.sync_copy(dst_hbm.at[pl.ds(w * ITERS, ITERS)], dst_v)

    def ones_body(i, carry):
        ones_v[pl.ds(i * 16, 16)] = jnp.ones((16,), jnp.float32)
        return carry
    lax.fori_loop(0, K // 16, ones_body, 0)

    def zero_body(i, carry):
        stage_v[pl.ds(i * 16, 16)] = jnp.zeros((16,), jnp.float32)
        return carry
    lax.fori_loop(0, ROWS_PER_TILE // 16, zero_body, 0)

    pltpu.sync_copy(stage_v, deg_sh.at[pl.ds(s * ROWS_PER_TILE, ROWS_PER_TILE)])
    plsc.subcore_barrier()

    def body(j, carry):
        pltpu.sync_copy(ones_v, deg_sh.at[dst_v.at[j]], add=True)
        return carry
    lax.fori_loop(0, ITERS, body, 0)

    plsc.subcore_barrier()
    pltpu.sync_copy(deg_sh.at[pl.ds(s * ROWS_PER_TILE, ROWS_PER_TILE)], stage_v)
    pltpu.sync_copy(stage_v, deg_out.at[c, pl.ds(s * ROWS_PER_TILE, ROWS_PER_TILE)])


@functools.lru_cache(maxsize=None)
def _deg_call():
    return pl.kernel(
        _deg_body,
        out_type=jax.ShapeDtypeStruct((NC, NP), jnp.float32),
        mesh=_mesh(),
        scratch_types=[
            pltpu.VMEM((ITERS, K), jnp.int32),
            pltpu.VMEM((K,), jnp.float32),
            pltpu.VMEM((ROWS_PER_TILE,), jnp.float32),
            pltpu.VMEM_SHARED((NP,), jnp.float32),
        ],
    )


def _prop_body(g_hbm, src_hbm, dst_hbm, s_out, src_v, dst_v, buf0, s_sh, sem0):
    c = lax.axis_index("c")
    s = lax.axis_index("s")
    w = c * NS + s

    pltpu.sync_copy(src_hbm.at[pl.ds(w * ITERS, ITERS)], src_v)
    pltpu.sync_copy(dst_hbm.at[pl.ds(w * ITERS, ITERS)], dst_v)

    _zero_vmem_2d(buf0, K, D)
    for r in range(ROWS_PER_TILE // K):
        pltpu.sync_copy(buf0, s_sh.at[pl.ds(s * ROWS_PER_TILE + r * K, K)])
    plsc.subcore_barrier()

    def body(j, carry):
        pltpu.async_copy(g_hbm.at[src_v.at[j]], buf0, sem0).wait()
        pltpu.sync_copy(buf0, s_sh.at[dst_v.at[j]], add=True)
        return carry

    lax.fori_loop(0, ITERS, body, 0)

    plsc.subcore_barrier()
    for r in range(ROWS_PER_TILE // K):
        off = s * ROWS_PER_TILE + r * K
        pltpu.sync_copy(s_sh.at[pl.ds(off, K)], buf0)
        pltpu.sync_copy(buf0, s_out.at[c, pl.ds(off, K)])


@functools.lru_cache(maxsize=None)
def _prop_call():
    return pl.kernel(
        _prop_body,
        out_type=jax.ShapeDtypeStruct((NC, NP, D), jnp.float32),
        mesh=_mesh(),
        scratch_types=[
            pltpu.VMEM((ITERS, K), jnp.int32),
            pltpu.VMEM((ITERS, K), jnp.int32),
            pltpu.VMEM((K, D), jnp.float32),
            pltpu.VMEM_SHARED((NP, D), jnp.float32),
            pltpu.SemaphoreType.DMA,
        ],
    )


R = 1024


def _dinv(dA, dB):
    return lax.rsqrt(dA[...] + dB[...] + 1.0)


def _l1_body(x, W, dA, dB, o):
    o[...] = _dinv(dA, dB) * jnp.dot(
        x[...], W[...], preferred_element_type=jnp.float32
    )


def _mid_body(sA, sB, g, dA, dB, b, W, o):
    dinv = _dinv(dA, dB)
    h = jnp.maximum(dinv * (sA[...] + sB[...] + g[...]) + b[...], 0.0)
    o[...] = dinv * jnp.dot(h, W[...], preferred_element_type=jnp.float32)


def _fin_body(sA, sB, g, dA, dB, b, W, bl, o):
    dinv = _dinv(dA, dB)
    h = jnp.maximum(dinv * (sA[...] + sB[...] + g[...]) + b[...], 0.0)
    o[...] = jnp.dot(h, W[...], preferred_element_type=jnp.float32) + bl[...]


_rows = pl.BlockSpec((R, D), lambda i: (i, 0))
_full = pl.BlockSpec((D, D), lambda i: (0, 0))
_col = pl.BlockSpec((R, 1), lambda i: (i, 0))
_row1 = pl.BlockSpec((1, D), lambda i: (0, 0))
_ospec = pl.BlockSpec((R, D), lambda i: (i, 0))
_oshape = jax.ShapeDtypeStruct((NP, D), jnp.float32)
_grid = (NP // R,)

_l1_call = pl.pallas_call(
    _l1_body, grid=_grid, out_shape=_oshape,
    in_specs=[_rows, _full, _col, _col], out_specs=_ospec,
)
_mid_call = pl.pallas_call(
    _mid_body, grid=_grid, out_shape=_oshape,
    in_specs=[_rows, _rows, _rows, _col, _col, _row1, _full], out_specs=_ospec,
)
_fin_call = pl.pallas_call(
    _fin_body, grid=_grid, out_shape=_oshape,
    in_specs=[_rows, _rows, _rows, _col, _col, _row1, _full, _row1],
    out_specs=_ospec,
)


def kernel(x, edge_index, W1, b1, W2, b2, W3, b3, Wl, bl):
    ei = edge_index.astype(jnp.int32)
    pad = EP - E
    srcp = jnp.concatenate([ei[0], jnp.zeros((pad,), jnp.int32)])
    dstp = jnp.concatenate([ei[1], jnp.full((pad,), N, jnp.int32)])
    src2 = srcp.reshape(NW * ITERS, K)
    dst2 = dstp.reshape(NW * ITERS, K)

    xp = jnp.pad(x, ((0, NP - N), (0, 0)))
    b1r = b1.reshape(1, D)
    b2r = b2.reshape(1, D)
    b3r = b3.reshape(1, D)
    blr = bl.reshape(1, D)

    deg = _deg_call()(dst2)
    dA = deg[0].reshape(NP, 1)
    dB = deg[1].reshape(NP, 1)

    g1 = _l1_call(xp, W1, dA, dB)
    s1 = _prop_call()(g1, src2, dst2)
    g2 = _mid_call(s1[0], s1[1], g1, dA, dB, b1r, W2)
    s2 = _prop_call()(g2, src2, dst2)
    g3 = _mid_call(s2[0], s2[1], g2, dA, dB, b2r, W3)
    s3 = _prop_call()(g3, src2, dst2)
    out = _fin_call(s3[0], s3[1], g3, dA, dB, b3r, Wl, blr)
    return out[:N]


# speedup vs baseline: 24.0128x; 3.6005x over previous
import functools

import jax
import jax.numpy as jnp
from jax import lax
from jax.experimental import pallas as pl
from jax.experimental.pallas import tpu as pltpu
from jax.experimental.pallas import tpu_sc as plsc

N = 10000
D = 128
E = 320000

NC = 2
NS = 16
NW = NC * NS

NP = 10240
ROWS_PER_TILE = NP // NS

K = 125
EPW = E // NW
ITERS = EPW // K
G = 8
GROUPS = ITERS // G
NIB = 3

def _zero_vmem_2d(ref, rows, cols):
    z = jnp.zeros((16,), jnp.float32)
    def body(i, c):
        for k in range(cols // 16):
            ref[i, pl.ds(k * 16, 16)] = z
        return c
    lax.fori_loop(0, rows, body, 0)


def _mesh():
    return plsc.VectorSubcoreMesh(
        core_axis_name="c", subcore_axis_name="s", num_cores=NC, num_subcores=NS
    )


def _deg_body(dst_hbm, deg_out, dst_v, ones_v, stage_v, deg_sh):
    c = lax.axis_index("c")
    s = lax.axis_index("s")
    w = c * NS + s

    pltpu.sync_copy(dst_hbm.at[w], dst_v)

    def ones_body(i, carry):
        ones_v[pl.ds(i * 16, 16)] = jnp.ones((16,), jnp.float32)
        return carry
    lax.fori_loop(0, 8, ones_body, 0)

    def zero_body(i, carry):
        stage_v[pl.ds(i * 16, 16)] = jnp.zeros((16,), jnp.float32)
        return carry
    lax.fori_loop(0, ROWS_PER_TILE // 16, zero_body, 0)

    pltpu.sync_copy(stage_v, deg_sh.at[pl.ds(s * ROWS_PER_TILE, ROWS_PER_TILE)])
    plsc.subcore_barrier()

    def body(j, carry):
        pltpu.sync_copy(ones_v.at[pl.ds(0, K)], deg_sh.at[dst_v.at[j]], add=True)
        return carry
    lax.fori_loop(0, ITERS, body, 0)

    plsc.subcore_barrier()
    pltpu.sync_copy(deg_sh.at[pl.ds(s * ROWS_PER_TILE, ROWS_PER_TILE)], stage_v)
    pltpu.sync_copy(stage_v, deg_out.at[c, pl.ds(s * ROWS_PER_TILE, ROWS_PER_TILE)])


@functools.lru_cache(maxsize=None)
def _deg_call():
    return pl.kernel(
        _deg_body,
        out_type=jax.ShapeDtypeStruct((NC, NP), jnp.float32),
        mesh=_mesh(),
        scratch_types=[
            pltpu.VMEM((ITERS, K), jnp.int32),
            pltpu.VMEM((128,), jnp.float32),
            pltpu.VMEM((ROWS_PER_TILE,), jnp.float32),
            pltpu.VMEM_SHARED((NP,), jnp.float32),
        ],
    )


def _prop_body(g_hbm, src_hbm, dst_hbm, s_out,
               srcI0, dstI0, srcI1, dstI1, srcI2, dstI2,
               bufA, bufB, s_sh, semGA, semGB, semSA, semSB, semI):
    c = lax.axis_index("c")
    s = lax.axis_index("s")
    w = c * NS + s
    srcs = [srcI0, srcI1, srcI2]
    dsts = [dstI0, dstI1, dstI2]
    bufs = [bufA, bufB]
    semG = [semGA, semGB]
    semS = [semSA, semSB]

    _zero_vmem_2d(bufA, 128, D)
    for r in range(ROWS_PER_TILE // 128):
        pltpu.sync_copy(bufA, s_sh.at[pl.ds(s * ROWS_PER_TILE + r * 128, 128)])
    plsc.subcore_barrier()

    def rows(j):
        return bufs[j % 2].at[pl.ds(0, K)]

    def load_idx(g, sync):
        sb, db = srcs[g % NIB], dsts[g % NIB]
        if sync:
            pltpu.sync_copy(src_hbm.at[w, pl.ds(g * G, G)], sb)
            pltpu.sync_copy(dst_hbm.at[w, pl.ds(g * G, G)], db)
        else:
            pltpu.async_copy(src_hbm.at[w, pl.ds(g * G, G)], sb, semI)
            pltpu.async_copy(dst_hbm.at[w, pl.ds(g * G, G)], db, semI)

    def wait_idx(g):
        sb, db = srcs[g % NIB], dsts[g % NIB]
        pltpu.make_async_copy(src_hbm.at[w, pl.ds(g * G, G)], sb, semI).wait()
        pltpu.make_async_copy(dst_hbm.at[w, pl.ds(g * G, G)], db, semI).wait()

    def gather_start(j):
        g, k = divmod(j, G)
        pltpu.async_copy(g_hbm.at[srcs[g % NIB].at[k]], rows(j), semG[j % 2])

    def gather_wait(j):
        g, k = divmod(j, G)
        pltpu.make_async_copy(
            g_hbm.at[srcs[g % NIB].at[k]], rows(j), semG[j % 2]).wait()

    def scat_start(j):
        g, k = divmod(j, G)
        pltpu.async_copy(
            rows(j), s_sh.at[dsts[g % NIB].at[k]], semS[j % 2], add=True)

    def scat_wait(j):
        g, k = divmod(j, G)
        pltpu.make_async_copy(
            rows(j), s_sh.at[dsts[g % NIB].at[k]], semS[j % 2]).wait()

    load_idx(0, sync=True)
    if GROUPS > 1:
        load_idx(1, sync=False)
    gather_start(0)
    for j in range(ITERS):
        gather_wait(j)
        scat_start(j)
        if j >= 1:
            scat_wait(j - 1)
        nj = j + 1
        if nj < ITERS:
            g1, k1 = divmod(nj, G)
            if k1 == 0:
                wait_idx(g1)
                if g1 + 1 < GROUPS:
                    load_idx(g1 + 1, sync=False)
            gather_start(nj)
    scat_wait(ITERS - 1)

    plsc.subcore_barrier()
    for r in range(ROWS_PER_TILE // 128):
        off = s * ROWS_PER_TILE + r * 128
        pltpu.sync_copy(s_sh.at[pl.ds(off, 128)], bufA)
        pltpu.sync_copy(bufA, s_out.at[c, pl.ds(off, 128)])


@functools.lru_cache(maxsize=None)
def _prop_call():
    return pl.kernel(
        _prop_body,
        out_type=jax.ShapeDtypeStruct((NC, NP, D), jnp.float32),
        mesh=_mesh(),
        scratch_types=(
            [pltpu.VMEM((G, K), jnp.int32)] * (2 * NIB)
            + [
                pltpu.VMEM((128, D), jnp.float32),
                pltpu.VMEM((128, D), jnp.float32),
                pltpu.VMEM_SHARED((NP, D), jnp.float32),
                pltpu.SemaphoreType.DMA,
                pltpu.SemaphoreType.DMA,
                pltpu.SemaphoreType.DMA,
                pltpu.SemaphoreType.DMA,
                pltpu.SemaphoreType.DMA,
            ]
        ),
    )


R = 1000


def _dinv(dA, dB):
    return lax.rsqrt(dA[...] + dB[...] + 1.0)


def _l1_body(x, W, dA, dB, o):
    o[...] = _dinv(dA, dB) * jnp.dot(
        x[...], W[...], preferred_element_type=jnp.float32
    )


def _mid_body(sA, sB, g, dA, dB, b, W, o):
    dinv = _dinv(dA, dB)
    h = jnp.maximum(dinv * (sA[...] + sB[...] + g[...]) + b[...], 0.0)
    o[...] = dinv * jnp.dot(h, W[...], preferred_element_type=jnp.float32)


def _fin_body(sA, sB, g, dA, dB, b, W, bl, o):
    dinv = _dinv(dA, dB)
    h = jnp.maximum(dinv * (sA[...] + sB[...] + g[...]) + b[...], 0.0)
    o[...] = jnp.dot(h, W[...], preferred_element_type=jnp.float32) + bl[...]


_rows = pl.BlockSpec((R, D), lambda i: (i, 0))
_full = pl.BlockSpec((D, D), lambda i: (0, 0))
_col = pl.BlockSpec((R, 1), lambda i: (i, 0))
_row1 = pl.BlockSpec((1, D), lambda i: (0, 0))
_ospec = pl.BlockSpec((R, D), lambda i: (i, 0))
_oshape = jax.ShapeDtypeStruct((N, D), jnp.float32)
_grid = (N // R,)

_l1_call = pl.pallas_call(
    _l1_body, grid=_grid, out_shape=_oshape,
    in_specs=[_rows, _full, _col, _col], out_specs=_ospec,
)
_mid_call = pl.pallas_call(
    _mid_body, grid=_grid, out_shape=_oshape,
    in_specs=[_rows, _rows, _rows, _col, _col, _row1, _full], out_specs=_ospec,
)
_fin_call = pl.pallas_call(
    _fin_body, grid=_grid, out_shape=_oshape,
    in_specs=[_rows, _rows, _rows, _col, _col, _row1, _full, _row1],
    out_specs=_ospec,
)


def kernel(x, edge_index, W1, b1, W2, b2, W3, b3, Wl, bl):
    ei = edge_index.astype(jnp.int32)
    src2 = ei[0].reshape(NW, ITERS, K)
    dst2 = ei[1].reshape(NW, ITERS, K)

    b1r = b1.reshape(1, D)
    b2r = b2.reshape(1, D)
    b3r = b3.reshape(1, D)
    blr = bl.reshape(1, D)

    deg = _deg_call()(dst2)
    dA = deg[0].reshape(NP, 1)
    dB = deg[1].reshape(NP, 1)

    g1 = _l1_call(x, W1, dA, dB)
    s1 = _prop_call()(g1, src2, dst2)
    g2 = _mid_call(s1[0], s1[1], g1, dA, dB, b1r, W2)
    s2 = _prop_call()(g2, src2, dst2)
    g3 = _mid_call(s2[0], s2[1], g2, dA, dB, b2r, W3)
    s3 = _prop_call()(g3, src2, dst2)
    return _fin_call(s3[0], s3[1], g3, dA, dB, b3r, Wl, blr)
